# baseline (speedup 1.0000x reference)
import jax
import jax.numpy as jnp
from jax.experimental import pallas as pl


def kernel(memory, node_ids, messages, W_ih, W_hh, b_ih, b_hh):
    N, D = memory.shape
    h = jnp.take(memory, node_ids, axis=0)
    gx = messages @ W_ih.T + b_ih
    gh = h @ W_hh.T + b_hh
    xr, xz, xn = gx[:, :D], gx[:, D:2 * D], gx[:, 2 * D:]
    hr, hz, hn = gh[:, :D], gh[:, D:2 * D], gh[:, 2 * D:]
    r = jax.nn.sigmoid(xr + hr)
    z = jax.nn.sigmoid(xz + hz)
    n = jnp.tanh(xn + r * hn)
    h_new = (1.0 - z) * n + z * h

    ids = node_ids.astype(jnp.int32)
    perm = jnp.argsort(ids, stable=True)
    sids = jnp.take(ids, perm)
    is_last = jnp.concatenate(
        [sids[1:] != sids[:-1], jnp.ones((1,), dtype=bool)])
    tgt = jnp.where(is_last, sids, N)
    new_memory = memory.at[tgt].set(jnp.take(h_new, perm, axis=0),
                                    mode="drop", unique_indices=True,
                                    indices_are_sorted=True)
    return new_memory


# speedup vs baseline: 1.5824x; 1.5824x over previous
import functools

import jax
import jax.numpy as jnp
from jax import lax
from jax.experimental import pallas as pl
from jax.experimental.pallas import tpu as pltpu
from jax.experimental.pallas import tpu_sc as plsc

NC, NS, NL = 2, 16, 16
NW = NC * NS


def _wid():
    return lax.axis_index("s") * NC + lax.axis_index("c")


def _sc_gather(table, idx):
    B = idx.shape[0]
    D = table.shape[1]
    bpw = B // NW

    def body(table_hbm, idx_hbm, out_hbm, idx_v, rows_v, sem):
        base = _wid() * bpw
        pltpu.sync_copy(idx_hbm.at[pl.ds(base, bpw)], idx_v)
        pltpu.async_copy(table_hbm.at[idx_v], rows_v, sem).wait()
        pltpu.sync_copy(rows_v, out_hbm.at[pl.ds(base, bpw)])

    return pl.kernel(
        body,
        out_type=jax.ShapeDtypeStruct((B, D), table.dtype),
        mesh=plsc.VectorSubcoreMesh(core_axis_name="c", subcore_axis_name="s"),
        scratch_types=[
            pltpu.VMEM((bpw,), jnp.int32),
            pltpu.VMEM((bpw, D), table.dtype),
            pltpu.SemaphoreType.DMA,
        ],
        name="sc_gather_rows",
    )(table, idx)


def _sc_scatter_inplace(mem_ref, rows, pos, ids):
    B = pos.shape[0]
    D = rows.shape[1]
    bpw = B // NW

    def body(mem_hbm, rows_hbm, pos_hbm, ids_hbm, pos_v, ids_v, rows_v, sem):
        base = _wid() * bpw
        pltpu.sync_copy(pos_hbm.at[pl.ds(base, bpw)], pos_v)
        pltpu.sync_copy(ids_hbm.at[pl.ds(base, bpw)], ids_v)
        pltpu.async_copy(rows_hbm.at[pos_v], rows_v, sem).wait()
        pltpu.sync_copy(rows_v, mem_hbm.at[ids_v])

    pl.kernel(
        body,
        out_type=(),
        mesh=plsc.VectorSubcoreMesh(core_axis_name="c", subcore_axis_name="s"),
        scratch_types=[
            pltpu.VMEM((bpw,), jnp.int32),
            pltpu.VMEM((bpw,), jnp.int32),
            pltpu.VMEM((bpw, D), rows.dtype),
            pltpu.SemaphoreType.DMA,
        ],
        name="sc_scatter_rows",
    )(mem_ref, rows, pos, ids)


def _tc_gru(h, messages, W_ih, W_hh, b_ih, b_hh, block_m=2048):
    B, D = h.shape
    cdims = (((1,), (1,)), ((), ()))

    def body(msg_ref, h_ref, wih_ref, whh_ref, bih_ref, bhh_ref, out_ref):
        msg = msg_ref[...]
        hh = h_ref[...]
        gx = lax.dot_general(msg, wih_ref[...], cdims,
                             preferred_element_type=jnp.float32) + bih_ref[...]
        gh = lax.dot_general(hh, whh_ref[...], cdims,
                             preferred_element_type=jnp.float32) + bhh_ref[...]
        xr, xz, xn = gx[:, :D], gx[:, D:2 * D], gx[:, 2 * D:]
        hr, hz, hn = gh[:, :D], gh[:, D:2 * D], gh[:, 2 * D:]
        r = jax.nn.sigmoid(xr + hr)
        z = jax.nn.sigmoid(xz + hz)
        n = jnp.tanh(xn + r * hn)
        out_ref[...] = (1.0 - z) * n + z * hh

    return pl.pallas_call(
        body,
        out_shape=jax.ShapeDtypeStruct((B, D), jnp.float32),
        grid=(B // block_m,),
        in_specs=[
            pl.BlockSpec((block_m, D), lambda i: (i, 0)),
            pl.BlockSpec((block_m, D), lambda i: (i, 0)),
            pl.BlockSpec(W_ih.shape, lambda i: (0, 0)),
            pl.BlockSpec(W_hh.shape, lambda i: (0, 0)),
            pl.BlockSpec((1, 3 * D), lambda i: (0, 0)),
            pl.BlockSpec((1, 3 * D), lambda i: (0, 0)),
        ],
        out_specs=pl.BlockSpec((block_m, D), lambda i: (i, 0)),
        name="tc_gru_cell",
    )(messages, h, W_ih, W_hh, b_ih.reshape(1, -1), b_hh.reshape(1, -1))


def _winners(ids):
    B = ids.shape[0]
    perm = jnp.argsort(ids, stable=True).astype(jnp.int32)
    sids = jnp.take(ids, perm)
    is_last = jnp.concatenate(
        [sids[1:] != sids[:-1], jnp.ones((1,), dtype=bool)])
    win_pos = jnp.where(is_last, perm, perm[B - 1])
    win_ids = jnp.where(is_last, sids, sids[B - 1])
    return win_pos, win_ids


def kernel(memory, node_ids, messages, W_ih, W_hh, b_ih, b_hh):
    ids = node_ids.astype(jnp.int32)
    h = _sc_gather(memory, ids)
    h_new = _tc_gru(h, messages, W_ih, W_hh, b_ih, b_hh)
    win_pos, win_ids = _winners(ids)
    mem_ref = jax.new_ref(memory)
    _sc_scatter_inplace(mem_ref, h_new, win_pos, win_ids)
    return jax.freeze(mem_ref)


# speedup vs baseline: 2.0561x; 1.2994x over previous
import functools

import jax
import jax.numpy as jnp
from jax import lax
from jax.experimental import pallas as pl
from jax.experimental.pallas import tpu as pltpu
from jax.experimental.pallas import tpu_sc as plsc

NC, NS, NL = 2, 16, 16
NW = NC * NS


def _wid():
    return lax.axis_index("s") * NC + lax.axis_index("c")


def _sc_gather(table, idx):
    B = idx.shape[0]
    D = table.shape[1]
    bpw = B // NW

    def body(table_hbm, idx_hbm, out_hbm, idx_v, rows_v, sem):
        base = _wid() * bpw
        pltpu.sync_copy(idx_hbm.at[pl.ds(base, bpw)], idx_v)
        pltpu.async_copy(table_hbm.at[idx_v], rows_v, sem).wait()
        pltpu.sync_copy(rows_v, out_hbm.at[pl.ds(base, bpw)])

    return pl.kernel(
        body,
        out_type=jax.ShapeDtypeStruct((B, D), table.dtype),
        mesh=plsc.VectorSubcoreMesh(core_axis_name="c", subcore_axis_name="s"),
        scratch_types=[
            pltpu.VMEM((bpw,), jnp.int32),
            pltpu.VMEM((bpw, D), table.dtype),
            pltpu.SemaphoreType.DMA,
        ],
        name="sc_gather_rows",
    )(table, idx)


def _sc_scatter_inplace(mem_ref, rows, winner_tab, ids):
    B = ids.shape[0]
    D = rows.shape[1]
    bpw = B // NW

    def body(mem_hbm, rows_hbm, tab_hbm, ids_hbm, pos_v, ids_v, rows_v, sem):
        base = _wid() * bpw
        pltpu.sync_copy(ids_hbm.at[pl.ds(base, bpw)], ids_v)
        pltpu.sync_copy(tab_hbm.at[ids_v], pos_v)
        pltpu.async_copy(rows_hbm.at[pos_v], rows_v, sem).wait()
        pltpu.sync_copy(rows_v, mem_hbm.at[ids_v])

    pl.kernel(
        body,
        out_type=(),
        mesh=plsc.VectorSubcoreMesh(core_axis_name="c", subcore_axis_name="s"),
        scratch_types=[
            pltpu.VMEM((bpw,), jnp.int32),
            pltpu.VMEM((bpw,), jnp.int32),
            pltpu.VMEM((bpw, D), rows.dtype),
            pltpu.SemaphoreType.DMA,
        ],
        name="sc_scatter_rows",
    )(mem_ref, rows, winner_tab, ids)


def _tc_gru(h, messages, W_ih, W_hh, b_ih, b_hh, block_m=2048):
    B, D = h.shape
    cdims = (((1,), (1,)), ((), ()))

    def body(msg_ref, h_ref, wih_ref, whh_ref, bih_ref, bhh_ref, out_ref):
        msg = msg_ref[...]
        hh = h_ref[...]
        gx = lax.dot_general(msg, wih_ref[...], cdims,
                             preferred_element_type=jnp.float32) + bih_ref[...]
        gh = lax.dot_general(hh, whh_ref[...], cdims,
                             preferred_element_type=jnp.float32) + bhh_ref[...]
        xr, xz, xn = gx[:, :D], gx[:, D:2 * D], gx[:, 2 * D:]
        hr, hz, hn = gh[:, :D], gh[:, D:2 * D], gh[:, 2 * D:]
        r = jax.nn.sigmoid(xr + hr)
        z = jax.nn.sigmoid(xz + hz)
        n = jnp.tanh(xn + r * hn)
        out_ref[...] = (1.0 - z) * n + z * hh

    return pl.pallas_call(
        body,
        out_shape=jax.ShapeDtypeStruct((B, D), jnp.float32),
        grid=(B // block_m,),
        in_specs=[
            pl.BlockSpec((block_m, D), lambda i: (i, 0)),
            pl.BlockSpec((block_m, D), lambda i: (i, 0)),
            pl.BlockSpec(W_ih.shape, lambda i: (0, 0)),
            pl.BlockSpec(W_hh.shape, lambda i: (0, 0)),
            pl.BlockSpec((1, 3 * D), lambda i: (0, 0)),
            pl.BlockSpec((1, 3 * D), lambda i: (0, 0)),
        ],
        out_specs=pl.BlockSpec((block_m, D), lambda i: (i, 0)),
        name="tc_gru_cell",
    )(messages, h, W_ih, W_hh, b_ih.reshape(1, -1), b_hh.reshape(1, -1))


def _winner_table(ids, n_nodes):
    B = ids.shape[0]
    return jnp.zeros((n_nodes,), jnp.int32).at[ids].max(
        jnp.arange(B, dtype=jnp.int32))


def kernel(memory, node_ids, messages, W_ih, W_hh, b_ih, b_hh):
    ids = node_ids.astype(jnp.int32)
    h = _sc_gather(memory, ids)
    h_new = _tc_gru(h, messages, W_ih, W_hh, b_ih, b_hh)
    winner_tab = _winner_table(ids, memory.shape[0])
    mem_ref = jax.new_ref(memory)
    _sc_scatter_inplace(mem_ref, h_new, winner_tab, ids)
    return jax.freeze(mem_ref)


# speedup vs baseline: 2.8934x; 1.4072x over previous
import dataclasses
import functools

import jax
import jax.numpy as jnp
from jax import lax
from jax.experimental import pallas as pl
from jax.experimental.pallas import tpu as pltpu
from jax.experimental.pallas import tpu_sc as plsc

NC, NS, NL = 2, 16, 16
NW = NC * NS


def _sc_compiler_params():
    cp = pltpu.CompilerParams()
    if "needs_layout_passes" in pltpu.CompilerParams.__dataclass_fields__:
        cp = dataclasses.replace(cp, needs_layout_passes=False)
    return cp


def _wid():
    return lax.axis_index("s") * NC + lax.axis_index("c")


def _sc_gather_and_winners(table, idx, own):
    B = idx.shape[0]
    D = table.shape[1]
    bpw = B // NW
    n_vregs = B // NL
    pos_bits = 14
    assert B <= (1 << pos_bits)

    def body(table_hbm, idx_hbm, rows_out, tab_out, idx_v, rows_v, tab_v, sem):
        w = _wid()
        base = w * bpw
        pltpu.sync_copy(idx_hbm, idx_v)
        gather_dma = pltpu.async_copy(
            table_hbm.at[idx_v.at[pl.ds(base, bpw)]], rows_v, sem)

        lo = w * own
        iota = lax.iota(jnp.int32, NL)
        shift = jnp.minimum(iota + 1, NL - 1)
        lane_last = iota == NL - 1
        shift_dn = lax.GatherDimensionNumbers(
            offset_dims=(), collapsed_slice_dims=(0,), start_index_map=(0,))

        @pl.loop(0, n_vregs)
        def _(c):
            ids16 = idx_v[pl.ds(c * NL, NL)]
            pos16 = c * NL + iota
            comb = lax.sort((ids16 << pos_bits) | pos16)
            ids_s = comb >> pos_bits
            pos_s = comb & ((1 << pos_bits) - 1)
            nxt = lax.gather(ids_s, shift[:, None], shift_dn, slice_sizes=(1,),
                             mode=lax.GatherScatterMode.PROMISE_IN_BOUNDS)
            run_last = (ids_s != nxt) | lane_last
            owned = (ids_s >= lo) & (ids_s < lo + own)
            plsc.store_scatter(tab_v, [ids_s - lo], pos_s,
                               mask=run_last & owned)

        pltpu.sync_copy(tab_v, tab_out.at[pl.ds(lo, own)])
        gather_dma.wait()
        pltpu.sync_copy(rows_v, rows_out.at[pl.ds(base, bpw)])

    return pl.kernel(
        body,
        out_type=(
            jax.ShapeDtypeStruct((B, D), table.dtype),
            jax.ShapeDtypeStruct((NW * own,), jnp.int32),
        ),
        mesh=plsc.VectorSubcoreMesh(core_axis_name="c", subcore_axis_name="s"),
        scratch_types=[
            pltpu.VMEM((B,), jnp.int32),
            pltpu.VMEM((bpw, D), table.dtype),
            pltpu.VMEM((own,), jnp.int32),
            pltpu.SemaphoreType.DMA,
        ],
        compiler_params=_sc_compiler_params(),
        name="sc_gather_rows",
    )(table, idx)


def _sc_scatter_inplace(mem_ref, rows, winner_tab, ids):
    B = ids.shape[0]
    D = rows.shape[1]
    bpw = B // NW

    def body(mem_hbm, rows_hbm, tab_hbm, ids_hbm, pos_v, ids_v, rows_v, sem):
        base = _wid() * bpw
        pltpu.sync_copy(ids_hbm.at[pl.ds(base, bpw)], ids_v)
        pltpu.sync_copy(tab_hbm.at[ids_v], pos_v)
        pltpu.async_copy(rows_hbm.at[pos_v], rows_v, sem).wait()
        pltpu.sync_copy(rows_v, mem_hbm.at[ids_v])

    pl.kernel(
        body,
        out_type=(),
        mesh=plsc.VectorSubcoreMesh(core_axis_name="c", subcore_axis_name="s"),
        scratch_types=[
            pltpu.VMEM((bpw,), jnp.int32),
            pltpu.VMEM((bpw,), jnp.int32),
            pltpu.VMEM((bpw, D), rows.dtype),
            pltpu.SemaphoreType.DMA,
        ],
        name="sc_scatter_rows",
    )(mem_ref, rows, winner_tab, ids)


def _tc_gru(h, messages, W_ih, W_hh, b_ih, b_hh, block_m=2048):
    B, D = h.shape
    cdims = (((1,), (1,)), ((), ()))

    def body(msg_ref, h_ref, wih_ref, whh_ref, bih_ref, bhh_ref, out_ref):
        msg = msg_ref[...]
        hh = h_ref[...]
        gx = lax.dot_general(msg, wih_ref[...], cdims,
                             preferred_element_type=jnp.float32) + bih_ref[...]
        gh = lax.dot_general(hh, whh_ref[...], cdims,
                             preferred_element_type=jnp.float32) + bhh_ref[...]
        xr, xz, xn = gx[:, :D], gx[:, D:2 * D], gx[:, 2 * D:]
        hr, hz, hn = gh[:, :D], gh[:, D:2 * D], gh[:, 2 * D:]
        r = jax.nn.sigmoid(xr + hr)
        z = jax.nn.sigmoid(xz + hz)
        n = jnp.tanh(xn + r * hn)
        out_ref[...] = (1.0 - z) * n + z * hh

    return pl.pallas_call(
        body,
        out_shape=jax.ShapeDtypeStruct((B, D), jnp.float32),
        grid=(B // block_m,),
        in_specs=[
            pl.BlockSpec((block_m, D), lambda i: (i, 0)),
            pl.BlockSpec((block_m, D), lambda i: (i, 0)),
            pl.BlockSpec(W_ih.shape, lambda i: (0, 0)),
            pl.BlockSpec(W_hh.shape, lambda i: (0, 0)),
            pl.BlockSpec((1, 3 * D), lambda i: (0, 0)),
            pl.BlockSpec((1, 3 * D), lambda i: (0, 0)),
        ],
        out_specs=pl.BlockSpec((block_m, D), lambda i: (i, 0)),
        name="tc_gru_cell",
    )(messages, h, W_ih, W_hh, b_ih.reshape(1, -1), b_hh.reshape(1, -1))


def kernel(memory, node_ids, messages, W_ih, W_hh, b_ih, b_hh):
    ids = node_ids.astype(jnp.int32)
    own = ((memory.shape[0] + NW - 1) // NW + 7) // 8 * 8
    h, winner_tab = _sc_gather_and_winners(memory, ids, own)
    h_new = _tc_gru(h, messages, W_ih, W_hh, b_ih, b_hh)
    mem_ref = jax.new_ref(memory)
    _sc_scatter_inplace(mem_ref, h_new, winner_tab, ids)
    return jax.freeze(mem_ref)


# speedup vs baseline: 2.9001x; 1.0023x over previous
import dataclasses
import functools

import jax
import jax.numpy as jnp
from jax import lax
from jax.experimental import pallas as pl
from jax.experimental.pallas import tpu as pltpu
from jax.experimental.pallas import tpu_sc as plsc

NC, NS, NL = 2, 16, 16
NW = NC * NS


def _sc_compiler_params():
    cp = pltpu.CompilerParams()
    if "needs_layout_passes" in pltpu.CompilerParams.__dataclass_fields__:
        cp = dataclasses.replace(cp, needs_layout_passes=False)
    return cp


def _wid():
    return lax.axis_index("s") * NC + lax.axis_index("c")


def _sc_gather_and_winners(table, idx, own):
    B = idx.shape[0]
    D = table.shape[1]
    bpw = B // NW
    n_vregs = B // NL
    pos_bits = 14
    assert B <= (1 << pos_bits)

    def body(table_hbm, idx_hbm, rows_out, tab_out, idx_v, rows_v, tab_v, sem):
        w = _wid()
        base = w * bpw
        pltpu.sync_copy(idx_hbm, idx_v)
        gather_dma = pltpu.async_copy(
            table_hbm.at[idx_v.at[pl.ds(base, bpw)]], rows_v, sem)

        lo = w * own
        iota = lax.iota(jnp.int32, NL)
        shift = jnp.minimum(iota + 1, NL - 1)
        lane_last = iota == NL - 1
        shift_dn = lax.GatherDimensionNumbers(
            offset_dims=(), collapsed_slice_dims=(0,), start_index_map=(0,))

        @pl.loop(0, n_vregs)
        def _(c):
            ids16 = idx_v[pl.ds(c * NL, NL)]
            pos16 = c * NL + iota
            comb = lax.sort((ids16 << pos_bits) | pos16)
            ids_s = comb >> pos_bits
            pos_s = comb & ((1 << pos_bits) - 1)
            nxt = lax.gather(ids_s, shift[:, None], shift_dn, slice_sizes=(1,),
                             mode=lax.GatherScatterMode.PROMISE_IN_BOUNDS)
            run_last = (ids_s != nxt) | lane_last
            owned = (ids_s >= lo) & (ids_s < lo + own)
            plsc.store_scatter(tab_v, [ids_s - lo], pos_s,
                               mask=run_last & owned)

        pltpu.sync_copy(tab_v, tab_out.at[pl.ds(lo, own)])
        gather_dma.wait()
        pltpu.sync_copy(rows_v, rows_out.at[pl.ds(base, bpw)])

    return pl.kernel(
        body,
        out_type=(
            jax.ShapeDtypeStruct((B, D), table.dtype),
            jax.ShapeDtypeStruct((NW * own,), jnp.int32),
        ),
        mesh=plsc.VectorSubcoreMesh(core_axis_name="c", subcore_axis_name="s"),
        scratch_types=[
            pltpu.VMEM((B,), jnp.int32),
            pltpu.VMEM((bpw, D), table.dtype),
            pltpu.VMEM((own,), jnp.int32),
            pltpu.SemaphoreType.DMA,
        ],
        compiler_params=_sc_compiler_params(),
        cost_estimate=pl.CostEstimate(
            flops=0, transcendentals=0, bytes_accessed=64 * 1024 * 1024),
        name="sc_gather_rows",
    )(table, idx)


def _sc_scatter_inplace(mem_ref, rows, winner_tab, ids):
    B = ids.shape[0]
    D = rows.shape[1]
    bpw = B // NW

    def body(mem_hbm, rows_hbm, tab_hbm, ids_hbm, pos_v, ids_v, rows_v, sem):
        base = _wid() * bpw
        pltpu.sync_copy(ids_hbm.at[pl.ds(base, bpw)], ids_v)
        pltpu.sync_copy(tab_hbm.at[ids_v], pos_v)
        pltpu.async_copy(rows_hbm.at[pos_v], rows_v, sem).wait()
        pltpu.sync_copy(rows_v, mem_hbm.at[ids_v])

    pl.kernel(
        body,
        out_type=(),
        mesh=plsc.VectorSubcoreMesh(core_axis_name="c", subcore_axis_name="s"),
        scratch_types=[
            pltpu.VMEM((bpw,), jnp.int32),
            pltpu.VMEM((bpw,), jnp.int32),
            pltpu.VMEM((bpw, D), rows.dtype),
            pltpu.SemaphoreType.DMA,
        ],
        cost_estimate=pl.CostEstimate(
            flops=0, transcendentals=0, bytes_accessed=32 * 1024 * 1024),
        name="sc_scatter_rows",
    )(mem_ref, rows, winner_tab, ids)


def _tc_gru(h, messages, W_ih, W_hh, b_ih, b_hh, block_m=2048):
    B, D = h.shape
    cdims = (((1,), (1,)), ((), ()))

    def body(msg_ref, h_ref, wih_ref, whh_ref, bih_ref, bhh_ref, out_ref):
        msg = msg_ref[...]
        hh = h_ref[...]
        gx = lax.dot_general(msg, wih_ref[...], cdims,
                             preferred_element_type=jnp.float32) + bih_ref[...]
        gh = lax.dot_general(hh, whh_ref[...], cdims,
                             preferred_element_type=jnp.float32) + bhh_ref[...]
        xr, xz, xn = gx[:, :D], gx[:, D:2 * D], gx[:, 2 * D:]
        hr, hz, hn = gh[:, :D], gh[:, D:2 * D], gh[:, 2 * D:]
        r = jax.nn.sigmoid(xr + hr)
        z = jax.nn.sigmoid(xz + hz)
        n = jnp.tanh(xn + r * hn)
        out_ref[...] = (1.0 - z) * n + z * hh

    return pl.pallas_call(
        body,
        out_shape=jax.ShapeDtypeStruct((B, D), jnp.float32),
        grid=(B // block_m,),
        in_specs=[
            pl.BlockSpec((block_m, D), lambda i: (i, 0)),
            pl.BlockSpec((block_m, D), lambda i: (i, 0)),
            pl.BlockSpec(W_ih.shape, lambda i: (0, 0)),
            pl.BlockSpec(W_hh.shape, lambda i: (0, 0)),
            pl.BlockSpec((1, 3 * D), lambda i: (0, 0)),
            pl.BlockSpec((1, 3 * D), lambda i: (0, 0)),
        ],
        out_specs=pl.BlockSpec((block_m, D), lambda i: (i, 0)),
        name="tc_gru_cell",
    )(messages, h, W_ih, W_hh, b_ih.reshape(1, -1), b_hh.reshape(1, -1))


def kernel(memory, node_ids, messages, W_ih, W_hh, b_ih, b_hh):
    ids = node_ids.astype(jnp.int32)
    own = ((memory.shape[0] + NW - 1) // NW + 7) // 8 * 8
    h, winner_tab = _sc_gather_and_winners(memory, ids, own)
    h_new = _tc_gru(h, messages, W_ih, W_hh, b_ih, b_hh)
    mem_ref = jax.new_ref(memory)
    _sc_scatter_inplace(mem_ref, h_new, winner_tab, ids)
    return jax.freeze(mem_ref)


# speedup vs baseline: 2.9104x; 1.0035x over previous
import dataclasses
import functools

import jax
import jax.numpy as jnp
from jax import lax
from jax.experimental import pallas as pl
from jax.experimental.pallas import tpu as pltpu
from jax.experimental.pallas import tpu_sc as plsc

NC, NS, NL = 2, 16, 16
NW = NC * NS


def _sc_compiler_params():
    cp = pltpu.CompilerParams(use_tc_tiling_on_sc=True)
    if "needs_layout_passes" in pltpu.CompilerParams.__dataclass_fields__:
        cp = dataclasses.replace(cp, needs_layout_passes=False)
    return cp


def _wid():
    return lax.axis_index("s") * NC + lax.axis_index("c")


def _sc_gather_and_winners(table, idx, own):
    B = idx.shape[0]
    D = table.shape[1]
    bpw = B // NW
    n_vregs = B // NL
    pos_bits = 14
    assert B <= (1 << pos_bits)

    def body(table_hbm, idx_hbm, rows_out, tab_out, idx_v, rows_v, tab_v, sem):
        w = _wid()
        base = w * bpw
        pltpu.sync_copy(idx_hbm, idx_v)
        gather_dma = pltpu.async_copy(
            table_hbm.at[idx_v.at[pl.ds(base, bpw)]], rows_v, sem)

        lo = w * own
        iota = lax.iota(jnp.int32, NL)
        shift = jnp.minimum(iota + 1, NL - 1)
        lane_last = iota == NL - 1
        shift_dn = lax.GatherDimensionNumbers(
            offset_dims=(), collapsed_slice_dims=(0,), start_index_map=(0,))

        @pl.loop(0, n_vregs)
        def _(c):
            ids16 = idx_v[pl.ds(c * NL, NL)]
            pos16 = c * NL + iota
            comb = lax.sort((ids16 << pos_bits) | pos16)
            ids_s = comb >> pos_bits
            pos_s = comb & ((1 << pos_bits) - 1)
            nxt = lax.gather(ids_s, shift[:, None], shift_dn, slice_sizes=(1,),
                             mode=lax.GatherScatterMode.PROMISE_IN_BOUNDS)
            run_last = (ids_s != nxt) | lane_last
            owned = (ids_s >= lo) & (ids_s < lo + own)
            plsc.store_scatter(tab_v, [ids_s - lo], pos_s,
                               mask=run_last & owned)

        pltpu.sync_copy(tab_v, tab_out.at[pl.ds(lo, own)])
        gather_dma.wait()
        pltpu.sync_copy(rows_v, rows_out.at[pl.ds(base, bpw)])

    return pl.kernel(
        body,
        out_type=(
            jax.ShapeDtypeStruct((B, D), table.dtype),
            jax.ShapeDtypeStruct((NW * own,), jnp.int32),
        ),
        mesh=plsc.VectorSubcoreMesh(core_axis_name="c", subcore_axis_name="s"),
        scratch_types=[
            pltpu.VMEM((B,), jnp.int32),
            pltpu.VMEM((bpw, D), table.dtype),
            pltpu.VMEM((own,), jnp.int32),
            pltpu.SemaphoreType.DMA,
        ],
        compiler_params=_sc_compiler_params(),
        cost_estimate=pl.CostEstimate(
            flops=0, transcendentals=0, bytes_accessed=64 * 1024 * 1024),
        name="sc_gather_rows",
    )(table, idx)


def _sc_scatter_inplace(mem_ref, rows, winner_tab, ids):
    B = ids.shape[0]
    D = rows.shape[1]
    bpw = B // NW

    def body(mem_hbm, rows_hbm, tab_hbm, ids_hbm, pos_v, ids_v, rows_v, sem):
        base = _wid() * bpw
        pltpu.sync_copy(ids_hbm.at[pl.ds(base, bpw)], ids_v)
        pltpu.sync_copy(tab_hbm.at[ids_v], pos_v)
        pltpu.async_copy(rows_hbm.at[pos_v], rows_v, sem).wait()
        pltpu.sync_copy(rows_v, mem_hbm.at[ids_v])

    pl.kernel(
        body,
        out_type=(),
        mesh=plsc.VectorSubcoreMesh(core_axis_name="c", subcore_axis_name="s"),
        scratch_types=[
            pltpu.VMEM((bpw,), jnp.int32),
            pltpu.VMEM((bpw,), jnp.int32),
            pltpu.VMEM((bpw, D), rows.dtype),
            pltpu.SemaphoreType.DMA,
        ],
        compiler_params=_sc_compiler_params(),
        cost_estimate=pl.CostEstimate(
            flops=0, transcendentals=0, bytes_accessed=32 * 1024 * 1024),
        name="sc_scatter_rows",
    )(mem_ref, rows, winner_tab, ids)


def _tc_gru(h, messages, W_ih, W_hh, b_ih, b_hh, block_m=2048):
    B, D = h.shape
    cdims = (((1,), (1,)), ((), ()))

    def body(msg_ref, h_ref, wih_ref, whh_ref, bih_ref, bhh_ref, out_ref):
        msg = msg_ref[...]
        hh = h_ref[...]
        gx = lax.dot_general(msg, wih_ref[...], cdims,
                             preferred_element_type=jnp.float32) + bih_ref[...]
        gh = lax.dot_general(hh, whh_ref[...], cdims,
                             preferred_element_type=jnp.float32) + bhh_ref[...]
        xr, xz, xn = gx[:, :D], gx[:, D:2 * D], gx[:, 2 * D:]
        hr, hz, hn = gh[:, :D], gh[:, D:2 * D], gh[:, 2 * D:]
        r = jax.nn.sigmoid(xr + hr)
        z = jax.nn.sigmoid(xz + hz)
        n = jnp.tanh(xn + r * hn)
        out_ref[...] = (1.0 - z) * n + z * hh

    return pl.pallas_call(
        body,
        out_shape=jax.ShapeDtypeStruct((B, D), jnp.float32),
        grid=(B // block_m,),
        in_specs=[
            pl.BlockSpec((block_m, D), lambda i: (i, 0)),
            pl.BlockSpec((block_m, D), lambda i: (i, 0)),
            pl.BlockSpec(W_ih.shape, lambda i: (0, 0)),
            pl.BlockSpec(W_hh.shape, lambda i: (0, 0)),
            pl.BlockSpec((1, 3 * D), lambda i: (0, 0)),
            pl.BlockSpec((1, 3 * D), lambda i: (0, 0)),
        ],
        out_specs=pl.BlockSpec((block_m, D), lambda i: (i, 0)),
        name="tc_gru_cell",
    )(messages, h, W_ih, W_hh, b_ih.reshape(1, -1), b_hh.reshape(1, -1))


def kernel(memory, node_ids, messages, W_ih, W_hh, b_ih, b_hh):
    ids = node_ids.astype(jnp.int32)
    own = ((memory.shape[0] + NW - 1) // NW + 7) // 8 * 8
    h, winner_tab = _sc_gather_and_winners(memory, ids, own)
    h_new = _tc_gru(h, messages, W_ih, W_hh, b_ih, b_hh)
    mem_ref = jax.new_ref(memory)
    _sc_scatter_inplace(mem_ref, h_new, winner_tab, ids)
    return jax.freeze(mem_ref)
